# Initial kernel scaffold; baseline (speedup 1.0000x reference)
import jax
import jax.numpy as jnp
from jax.experimental import pallas as pl


def kernel(x, edge_index, W1, b1, W2, b2):
    raise NotImplementedError("write your pallas kernel here")


# speedup vs baseline: 48.6049x; 48.6049x over previous
import functools

import jax
import jax.numpy as jnp
from jax import lax
from jax.experimental import pallas as pl
from jax.experimental.pallas import tpu as pltpu
from jax.experimental.pallas import tpu_sc as plsc

N = 10000
E = 320000
NTILES = 16
PN = 10240
NPAD = PN - N
OWN = PN // NTILES
OWNV = OWN // 16
EPT = E // NTILES

_D_IN = 128
_H1 = 256
_H2 = 128


def _rsqrt16(d):
    x = jnp.maximum(d, 1.0)
    i = lax.bitcast_convert_type(x, jnp.int32)
    i = jnp.int32(0x5F3759DF) - lax.shift_right_logical(i, 1)
    y = lax.bitcast_convert_type(i, jnp.float32)
    for _ in range(3):
        y = y * (jnp.float32(1.5) - jnp.float32(0.5) * x * y * y)
    return jnp.where(d > 0.5, y, 0.0)


_MESH = plsc.VectorSubcoreMesh(
    core_axis_name="c", subcore_axis_name="s", num_cores=1
)


@functools.partial(
    pl.kernel,
    out_type=(
        jax.ShapeDtypeStruct((PN,), jnp.float32),
        jax.ShapeDtypeStruct((PN,), jnp.float32),
    ),
    mesh=_MESH,
    scratch_types=[
        pltpu.VMEM((EPT,), jnp.int32),
        pltpu.VMEM((EPT,), jnp.int32),
        pltpu.VMEM((EPT,), jnp.float32),
        pltpu.VMEM((OWN,), jnp.float32),
        pltpu.VMEM((OWN,), jnp.float32),
        pltpu.VMEM((OWN,), jnp.float32),
        pltpu.SemaphoreType.DMA,
        pltpu.VMEM_SHARED((PN,), jnp.float32),
        pltpu.VMEM_SHARED((PN,), jnp.float32),
    ],
)
def _sc_uw(src_hbm, dst_hbm, u_hbm, w_hbm,
           srcbuf, dstbuf, gbuf, own, normown, zbuf, sem, shacc, shtab):
    tid = lax.axis_index("s")
    own_sl = pl.ds(OWN * tid, OWN)
    edge_sl = pl.ds(EPT * tid, EPT)

    pltpu.sync_copy(src_hbm.at[edge_sl], srcbuf)
    pltpu.sync_copy(dst_hbm.at[edge_sl], dstbuf)

    def fill(ref, val):
        v = jnp.full((16,), val, jnp.float32)

        def body(j, c):
            ref[pl.ds(j * 16, 16)] = v
            return c

        lax.fori_loop(0, OWNV, body, 0)

    fill(zbuf, 0.0)
    pltpu.sync_copy(zbuf, shacc.at[own_sl])

    def fill_ones(j, c):
        gbuf[pl.ds(j * 16, 16)] = jnp.ones((16,), jnp.float32)
        return c

    lax.fori_loop(0, EPT // 16, fill_ones, 0)
    plsc.subcore_barrier()

    pltpu.sync_copy(gbuf, shacc.at[dstbuf], add=True)
    plsc.subcore_barrier()

    pltpu.sync_copy(shacc.at[own_sl], own)

    def norm_body(j, c):
        sl = pl.ds(j * 16, 16)
        normown[sl] = _rsqrt16(own[sl])
        return c

    lax.fori_loop(0, OWNV, norm_body, 0)
    pltpu.sync_copy(normown, shtab.at[own_sl])
    pltpu.sync_copy(zbuf, shacc.at[own_sl])
    plsc.subcore_barrier()

    def scale_own_by_norm():
        def body(j, c):
            sl = pl.ds(j * 16, 16)
            own[sl] = own[sl] * normown[sl]
            return c

        lax.fori_loop(0, OWNV, body, 0)

    pltpu.async_copy(shtab.at[dstbuf], gbuf, sem).wait()
    pltpu.sync_copy(gbuf, shacc.at[srcbuf], add=True)
    plsc.subcore_barrier()
    pltpu.sync_copy(shacc.at[own_sl], own)
    scale_own_by_norm()
    pltpu.sync_copy(own, u_hbm.at[own_sl])
    scale_own_by_norm()
    pltpu.sync_copy(own, shtab.at[own_sl])
    pltpu.sync_copy(zbuf, shacc.at[own_sl])
    plsc.subcore_barrier()

    pltpu.async_copy(shtab.at[dstbuf], gbuf, sem).wait()
    pltpu.sync_copy(gbuf, shacc.at[srcbuf], add=True)
    plsc.subcore_barrier()
    pltpu.sync_copy(shacc.at[own_sl], own)
    scale_own_by_norm()
    pltpu.sync_copy(own, w_hbm.at[own_sl])


def _tc_body(x_ref, w_ref, u_ref, w1_ref, b1_ref, w2_ref, b2_ref, o_ref):
    r = jnp.sum(x_ref[...] * w_ref[...], axis=0, keepdims=True)
    s = jnp.sum(u_ref[...])
    g = lax.dot_general(
        r, w1_ref[...], (((1,), (0,)), ((), ())),
        precision=lax.Precision.HIGHEST,
        preferred_element_type=jnp.float32,
    ) + s * b1_ref[...]
    o_ref[...] = lax.dot_general(
        g, w2_ref[...], (((1,), (0,)), ((), ())),
        precision=lax.Precision.HIGHEST,
        preferred_element_type=jnp.float32,
    ) + jnp.float32(N) * b2_ref[...]


_tc_head = pl.pallas_call(
    _tc_body,
    out_shape=jax.ShapeDtypeStruct((1, _H2), jnp.float32),
)


def kernel(x, edge_index, W1, b1, W2, b2):
    u_pn, w_pn = _sc_uw(edge_index[0], edge_index[1])
    u = u_pn[:N].reshape(N, 1)
    w = w_pn[:N].reshape(N, 1)
    return _tc_head(x, w, u, W1, b1.reshape(1, _H1), W2, b2.reshape(1, _H2))


# speedup vs baseline: 57.0273x; 1.1733x over previous
import functools

import jax
import jax.numpy as jnp
from jax import lax
from jax.experimental import pallas as pl
from jax.experimental.pallas import tpu as pltpu
from jax.experimental.pallas import tpu_sc as plsc

N = 10000
E = 320000
NTILES = 16
PN = 10240
NPAD = PN - N
OWN = PN // NTILES
OWNV = OWN // 16
EPT = E // NTILES

_D_IN = 128
_H1 = 256
_H2 = 128


def _rsqrt16(d):
    x = jnp.maximum(d, 1.0)
    i = lax.bitcast_convert_type(x, jnp.int32)
    i = jnp.int32(0x5F3759DF) - lax.shift_right_logical(i, 1)
    y = lax.bitcast_convert_type(i, jnp.float32)
    for _ in range(3):
        y = y * (jnp.float32(1.5) - jnp.float32(0.5) * x * y * y)
    return jnp.where(d > 0.5, y, 0.0)


_MESH = plsc.VectorSubcoreMesh(
    core_axis_name="c", subcore_axis_name="s", num_cores=1
)


@functools.partial(
    pl.kernel,
    out_type=(
        jax.ShapeDtypeStruct((PN,), jnp.float32),
        jax.ShapeDtypeStruct((PN,), jnp.float32),
    ),
    mesh=_MESH,
    scratch_types=[
        pltpu.VMEM((EPT,), jnp.int32),
        pltpu.VMEM((EPT,), jnp.int32),
        pltpu.VMEM((EPT,), jnp.float32),
        pltpu.VMEM((OWN,), jnp.float32),
        pltpu.VMEM((OWN,), jnp.float32),
        pltpu.VMEM((OWN,), jnp.float32),
        pltpu.SemaphoreType.DMA,
        pltpu.VMEM_SHARED((PN,), jnp.float32),
        pltpu.VMEM_SHARED((PN,), jnp.float32),
    ],
)
def _sc_uw(src_hbm, dst_hbm, u_hbm, w_hbm,
           srcbuf, dstbuf, gbuf, own, normown, zbuf, sem, shacc, shtab):
    tid = lax.axis_index("s")
    own_sl = pl.ds(OWN * tid, OWN)
    edge_sl = pl.ds(EPT * tid, EPT)

    pltpu.sync_copy(src_hbm.at[edge_sl], srcbuf)
    pltpu.sync_copy(dst_hbm.at[edge_sl], dstbuf)

    def fill(ref, val):
        v = jnp.full((16,), val, jnp.float32)

        def body(j, c):
            ref[pl.ds(j * 16, 16)] = v
            return c

        lax.fori_loop(0, OWNV, body, 0)

    fill(zbuf, 0.0)
    pltpu.sync_copy(zbuf, shacc.at[own_sl])

    def fill_ones(j, c):
        gbuf[pl.ds(j * 16, 16)] = jnp.ones((16,), jnp.float32)
        return c

    lax.fori_loop(0, EPT // 16, fill_ones, 0)
    plsc.subcore_barrier()

    pltpu.sync_copy(gbuf, shacc.at[dstbuf], add=True)
    plsc.subcore_barrier()

    pltpu.sync_copy(shacc.at[own_sl], own)

    def norm_body(j, c):
        sl = pl.ds(j * 16, 16)
        normown[sl] = _rsqrt16(own[sl])
        return c

    lax.fori_loop(0, OWNV, norm_body, 0)
    pltpu.sync_copy(normown, shtab.at[own_sl])
    pltpu.sync_copy(zbuf, shacc.at[own_sl])
    plsc.subcore_barrier()

    def scale_own_by_norm():
        def body(j, c):
            sl = pl.ds(j * 16, 16)
            own[sl] = own[sl] * normown[sl]
            return c

        lax.fori_loop(0, OWNV, body, 0)

    pltpu.async_copy(shtab.at[dstbuf], gbuf, sem).wait()
    pltpu.sync_copy(gbuf, shacc.at[srcbuf], add=True)
    plsc.subcore_barrier()
    pltpu.sync_copy(shacc.at[own_sl], own)
    scale_own_by_norm()
    pltpu.sync_copy(own, u_hbm.at[own_sl])
    scale_own_by_norm()
    pltpu.sync_copy(own, shtab.at[own_sl])
    pltpu.sync_copy(zbuf, shacc.at[own_sl])
    plsc.subcore_barrier()

    pltpu.async_copy(shtab.at[dstbuf], gbuf, sem).wait()
    pltpu.sync_copy(gbuf, shacc.at[srcbuf], add=True)
    plsc.subcore_barrier()
    pltpu.sync_copy(shacc.at[own_sl], own)
    scale_own_by_norm()
    pltpu.sync_copy(own, w_hbm.at[own_sl])


def _tc_body(x_ref, w_ref, u_ref, w1_ref, b1_ref, w2_ref, b2_ref, o_ref):
    wrow = w_ref[...].reshape(1, PN)[:, :N]
    r = lax.dot_general(
        wrow, x_ref[...], (((1,), (0,)), ((), ())),
        precision=lax.Precision.HIGHEST,
        preferred_element_type=jnp.float32,
    )
    s = jnp.sum(u_ref[...])
    g = lax.dot_general(
        r, w1_ref[...], (((1,), (0,)), ((), ())),
        precision=lax.Precision.HIGHEST,
        preferred_element_type=jnp.float32,
    ) + s * b1_ref[...].reshape(1, _H1)
    o_ref[...] = lax.dot_general(
        g, w2_ref[...], (((1,), (0,)), ((), ())),
        precision=lax.Precision.HIGHEST,
        preferred_element_type=jnp.float32,
    ) + jnp.float32(N) * b2_ref[...].reshape(1, _H2)


_tc_head = pl.pallas_call(
    _tc_body,
    out_shape=jax.ShapeDtypeStruct((1, _H2), jnp.float32),
)


def kernel(x, edge_index, W1, b1, W2, b2):
    u_pn, w_pn = _sc_uw(edge_index[0], edge_index[1])
    return _tc_head(x, w_pn, u_pn, W1, b1, W2, b2)


# speedup vs baseline: 65.8107x; 1.1540x over previous
import functools

import jax
import jax.numpy as jnp
from jax import lax
from jax.experimental import pallas as pl
from jax.experimental.pallas import tpu as pltpu
from jax.experimental.pallas import tpu_sc as plsc

N = 10000
E = 320000
NTILES = 16
PN = 10240
NPAD = PN - N
OWN = PN // NTILES
OWNV = OWN // 16
EPT = E // NTILES

_D_IN = 128
_H1 = 256
_H2 = 128


def _rsqrt16(d):
    x = jnp.maximum(d, 1.0)
    i = lax.bitcast_convert_type(x, jnp.int32)
    i = jnp.int32(0x5F3759DF) - lax.shift_right_logical(i, 1)
    y = lax.bitcast_convert_type(i, jnp.float32)
    for _ in range(3):
        y = y * (jnp.float32(1.5) - jnp.float32(0.5) * x * y * y)
    return jnp.where(d > 0.5, y, 0.0)


_MESH = plsc.VectorSubcoreMesh(
    core_axis_name="c", subcore_axis_name="s", num_cores=1
)


@functools.partial(
    pl.kernel,
    out_type=(
        jax.ShapeDtypeStruct((PN,), jnp.float32),
        jax.ShapeDtypeStruct((PN,), jnp.float32),
    ),
    mesh=_MESH,
    scratch_types=[
        pltpu.VMEM((EPT,), jnp.int32),
        pltpu.VMEM((EPT,), jnp.int32),
        pltpu.VMEM((EPT,), jnp.float32),
        pltpu.VMEM((OWN,), jnp.float32),
        pltpu.VMEM((OWN,), jnp.float32),
        pltpu.VMEM((OWN,), jnp.float32),
        pltpu.SemaphoreType.DMA,
        pltpu.VMEM_SHARED((PN,), jnp.float32),
        pltpu.VMEM_SHARED((PN,), jnp.float32),
    ],
)
def _sc_uw(edge_hbm, u_hbm, w_hbm,
           srcbuf, dstbuf, gbuf, own, normown, zbuf, sem, shacc, shtab):
    tid = lax.axis_index("s")
    own_sl = pl.ds(OWN * tid, OWN)

    pltpu.sync_copy(edge_hbm.at[pl.ds(EPT * tid, EPT)], srcbuf)
    pltpu.sync_copy(edge_hbm.at[pl.ds(E + EPT * tid, EPT)], dstbuf)

    def fill(ref, val):
        v = jnp.full((16,), val, jnp.float32)

        def body(j, c):
            ref[pl.ds(j * 16, 16)] = v
            return c

        lax.fori_loop(0, OWNV, body, 0)

    fill(zbuf, 0.0)
    pltpu.sync_copy(zbuf, shacc.at[own_sl])

    def fill_ones(j, c):
        gbuf[pl.ds(j * 16, 16)] = jnp.ones((16,), jnp.float32)
        return c

    lax.fori_loop(0, EPT // 16, fill_ones, 0)
    plsc.subcore_barrier()

    pltpu.sync_copy(gbuf, shacc.at[dstbuf], add=True)
    plsc.subcore_barrier()

    pltpu.sync_copy(shacc.at[own_sl], own)

    def norm_body(j, c):
        sl = pl.ds(j * 16, 16)
        normown[sl] = _rsqrt16(own[sl])
        return c

    lax.fori_loop(0, OWNV, norm_body, 0)
    pltpu.sync_copy(normown, shtab.at[own_sl])
    pltpu.sync_copy(zbuf, shacc.at[own_sl])
    plsc.subcore_barrier()

    def scale_own_by_norm():
        def body(j, c):
            sl = pl.ds(j * 16, 16)
            own[sl] = own[sl] * normown[sl]
            return c

        lax.fori_loop(0, OWNV, body, 0)

    pltpu.async_copy(shtab.at[dstbuf], gbuf, sem).wait()
    pltpu.sync_copy(gbuf, shacc.at[srcbuf], add=True)
    plsc.subcore_barrier()
    pltpu.sync_copy(shacc.at[own_sl], own)
    scale_own_by_norm()
    pltpu.sync_copy(own, u_hbm.at[own_sl])
    scale_own_by_norm()
    pltpu.sync_copy(own, shtab.at[own_sl])
    pltpu.sync_copy(zbuf, shacc.at[own_sl])
    plsc.subcore_barrier()

    pltpu.async_copy(shtab.at[dstbuf], gbuf, sem).wait()
    pltpu.sync_copy(gbuf, shacc.at[srcbuf], add=True)
    plsc.subcore_barrier()
    pltpu.sync_copy(shacc.at[own_sl], own)
    scale_own_by_norm()
    pltpu.sync_copy(own, w_hbm.at[own_sl])


def _tc_body(x_ref, w_ref, u_ref, w1_ref, b1_ref, w2_ref, b2_ref, o_ref):
    wrow = w_ref[...].reshape(1, PN)[:, :N]
    r = lax.dot_general(
        wrow, x_ref[...], (((1,), (0,)), ((), ())),
        precision=lax.Precision.HIGHEST,
        preferred_element_type=jnp.float32,
    )
    s = jnp.sum(u_ref[...])
    g = lax.dot_general(
        r, w1_ref[...], (((1,), (0,)), ((), ())),
        precision=lax.Precision.HIGHEST,
        preferred_element_type=jnp.float32,
    ) + s * b1_ref[...].reshape(1, _H1)
    o_ref[...] = lax.dot_general(
        g, w2_ref[...], (((1,), (0,)), ((), ())),
        precision=lax.Precision.HIGHEST,
        preferred_element_type=jnp.float32,
    ) + jnp.float32(N) * b2_ref[...].reshape(1, _H2)


_tc_head = pl.pallas_call(
    _tc_body,
    out_shape=jax.ShapeDtypeStruct((1, _H2), jnp.float32),
)


def kernel(x, edge_index, W1, b1, W2, b2):
    u_pn, w_pn = _sc_uw(edge_index.reshape(-1))
    return _tc_head(x, w_pn, u_pn, W1, b1, W2, b2)


# speedup vs baseline: 73.2833x; 1.1135x over previous
import functools

import jax
import jax.numpy as jnp
from jax import lax
from jax.experimental import pallas as pl
from jax.experimental.pallas import tpu as pltpu
from jax.experimental.pallas import tpu_sc as plsc

N = 10000
E = 320000
NC = 2
NTILES = 16
NW = NC * NTILES
PN = 10240
OWNT = PN // NTILES
OWNTV = OWNT // 16
EPW = E // NW

_D_IN = 128
_H1 = 256
_H2 = 128


def _rsqrt16(d):
    x = jnp.maximum(d, 1.0)
    i = lax.bitcast_convert_type(x, jnp.int32)
    i = jnp.int32(0x5F3759DF) - lax.shift_right_logical(i, 1)
    y = lax.bitcast_convert_type(i, jnp.float32)
    for _ in range(3):
        y = y * (jnp.float32(1.5) - jnp.float32(0.5) * x * y * y)
    return jnp.where(d > 0.5, y, 0.0)


_MESH = plsc.VectorSubcoreMesh(
    core_axis_name="c", subcore_axis_name="s", num_cores=NC
)


def _fill(ref, nvec, val):
    v = jnp.full((16,), val, jnp.float32)

    def body(j, carry):
        ref[pl.ds(j * 16, 16)] = v
        return carry

    lax.fori_loop(0, nvec, body, 0)


@functools.partial(
    pl.kernel,
    out_type=jax.ShapeDtypeStruct((NC * PN,), jnp.float32),
    mesh=_MESH,
    scratch_types=[
        pltpu.VMEM((EPW,), jnp.int32),
        pltpu.VMEM((EPW,), jnp.float32),
        pltpu.VMEM((OWNT,), jnp.float32),
        pltpu.SemaphoreType.DMA,
        pltpu.VMEM_SHARED((PN,), jnp.float32),
    ],
)
def _sc_deg(edge_hbm, degp_hbm, dstbuf, ones, zbuf, sem, shacc):
    cid = lax.axis_index("c")
    tid = lax.axis_index("s")
    wid = tid * NC + cid
    tile_sl = pl.ds(OWNT * tid, OWNT)

    cp = pltpu.async_copy(edge_hbm.at[pl.ds(E + EPW * wid, EPW)], dstbuf, sem)
    _fill(zbuf, OWNTV, 0.0)
    pltpu.sync_copy(zbuf, shacc.at[tile_sl])
    _fill(ones, EPW // 16, 1.0)
    cp.wait()
    plsc.subcore_barrier()

    pltpu.sync_copy(ones, shacc.at[dstbuf], add=True)
    plsc.subcore_barrier()
    pltpu.sync_copy(shacc.at[tile_sl], degp_hbm.at[pl.ds(PN * cid + OWNT * tid, OWNT)])


@functools.partial(
    pl.kernel,
    out_type=jax.ShapeDtypeStruct((NC * PN,), jnp.float32),
    mesh=_MESH,
    scratch_types=[
        pltpu.VMEM((EPW,), jnp.int32),
        pltpu.VMEM((EPW,), jnp.int32),
        pltpu.VMEM((EPW,), jnp.float32),
        pltpu.VMEM((OWNT,), jnp.float32),
        pltpu.VMEM((OWNT,), jnp.float32),
        pltpu.VMEM((OWNT,), jnp.float32),
        pltpu.VMEM((OWNT,), jnp.float32),
        pltpu.SemaphoreType.DMA,
        pltpu.VMEM_SHARED((PN,), jnp.float32),
        pltpu.VMEM_SHARED((PN,), jnp.float32),
    ],
)
def _sc_c(edge_hbm, degp_hbm, cp_hbm,
          srcbuf, dstbuf, gbuf, p0, p1, normt, zbuf, sem, shacc, shtab):
    cid = lax.axis_index("c")
    tid = lax.axis_index("s")
    wid = tid * NC + cid
    tile_sl = pl.ds(OWNT * tid, OWNT)

    cps = pltpu.async_copy(edge_hbm.at[pl.ds(EPW * wid, EPW)], srcbuf, sem)
    cpd = pltpu.async_copy(edge_hbm.at[pl.ds(E + EPW * wid, EPW)], dstbuf, sem)
    pltpu.sync_copy(degp_hbm.at[pl.ds(OWNT * tid, OWNT)], p0)
    pltpu.sync_copy(degp_hbm.at[pl.ds(PN + OWNT * tid, OWNT)], p1)
    _fill(zbuf, OWNTV, 0.0)
    pltpu.sync_copy(zbuf, shacc.at[tile_sl])

    def norm_body(j, carry):
        sl = pl.ds(j * 16, 16)
        normt[sl] = _rsqrt16(p0[sl] + p1[sl])
        return carry

    lax.fori_loop(0, OWNTV, norm_body, 0)
    pltpu.sync_copy(normt, shtab.at[tile_sl])
    cps.wait()
    cpd.wait()
    plsc.subcore_barrier()

    pltpu.async_copy(shtab.at[dstbuf], gbuf, sem).wait()
    pltpu.sync_copy(gbuf, shacc.at[srcbuf], add=True)
    plsc.subcore_barrier()
    pltpu.sync_copy(shacc.at[tile_sl], cp_hbm.at[pl.ds(PN * cid + OWNT * tid, OWNT)])


@functools.partial(
    pl.kernel,
    out_type=jax.ShapeDtypeStruct((NC * PN,), jnp.float32),
    mesh=_MESH,
    scratch_types=[
        pltpu.VMEM((EPW,), jnp.int32),
        pltpu.VMEM((EPW,), jnp.int32),
        pltpu.VMEM((EPW,), jnp.float32),
        pltpu.VMEM((OWNT,), jnp.float32),
        pltpu.VMEM((OWNT,), jnp.float32),
        pltpu.VMEM((OWNT,), jnp.float32),
        pltpu.VMEM((OWNT,), jnp.float32),
        pltpu.SemaphoreType.DMA,
        pltpu.VMEM_SHARED((PN,), jnp.float32),
        pltpu.VMEM_SHARED((PN,), jnp.float32),
    ],
)
def _sc_w(edge_hbm, degp_hbm, cp_hbm, wp_hbm,
          srcbuf, dstbuf, gbuf, p0, p1, tbuf, zbuf, sem, shacc, shtab):
    cid = lax.axis_index("c")
    tid = lax.axis_index("s")
    wid = tid * NC + cid
    tile_sl = pl.ds(OWNT * tid, OWNT)

    cps = pltpu.async_copy(edge_hbm.at[pl.ds(EPW * wid, EPW)], srcbuf, sem)
    cpd = pltpu.async_copy(edge_hbm.at[pl.ds(E + EPW * wid, EPW)], dstbuf, sem)
    pltpu.sync_copy(degp_hbm.at[pl.ds(OWNT * tid, OWNT)], p0)
    pltpu.sync_copy(degp_hbm.at[pl.ds(PN + OWNT * tid, OWNT)], p1)
    _fill(zbuf, OWNTV, 0.0)

    def norm_body(j, carry):
        sl = pl.ds(j * 16, 16)
        zbuf[sl] = _rsqrt16(p0[sl] + p1[sl])
        return carry

    lax.fori_loop(0, OWNTV, norm_body, 0)
    pltpu.sync_copy(cp_hbm.at[pl.ds(OWNT * tid, OWNT)], p0)
    pltpu.sync_copy(cp_hbm.at[pl.ds(PN + OWNT * tid, OWNT)], p1)

    def t_body(j, carry):
        sl = pl.ds(j * 16, 16)
        nrm = zbuf[sl]
        tbuf[sl] = nrm * nrm * (p0[sl] + p1[sl])
        return carry

    lax.fori_loop(0, OWNTV, t_body, 0)
    pltpu.sync_copy(tbuf, shtab.at[tile_sl])
    _fill(zbuf, OWNTV, 0.0)
    pltpu.sync_copy(zbuf, shacc.at[tile_sl])
    cps.wait()
    cpd.wait()
    plsc.subcore_barrier()

    pltpu.async_copy(shtab.at[dstbuf], gbuf, sem).wait()
    pltpu.sync_copy(gbuf, shacc.at[srcbuf], add=True)
    plsc.subcore_barrier()
    pltpu.sync_copy(shacc.at[tile_sl], wp_hbm.at[pl.ds(PN * cid + OWNT * tid, OWNT)])


def _tc_body(x_ref, degp_ref, cp_ref, wp_ref, w1_ref, b1_ref, w2_ref, b2_ref,
             o_ref):
    degp = degp_ref[...].reshape(NC, PN)
    cp = cp_ref[...].reshape(NC, PN)
    wp = wp_ref[...].reshape(NC, PN)
    deg = degp[0] + degp[1]
    norm = jnp.where(deg > 0, lax.rsqrt(jnp.maximum(deg, 1.0)), 0.0)
    u = norm * (cp[0] + cp[1])
    w = norm * (wp[0] + wp[1])
    wrow = w.reshape(1, PN)[:, :N]
    r = lax.dot_general(
        wrow, x_ref[...], (((1,), (0,)), ((), ())),
        precision=lax.Precision.HIGHEST,
        preferred_element_type=jnp.float32,
    )
    s = jnp.sum(u)
    g = lax.dot_general(
        r, w1_ref[...], (((1,), (0,)), ((), ())),
        precision=lax.Precision.HIGHEST,
        preferred_element_type=jnp.float32,
    ) + s * b1_ref[...].reshape(1, _H1)
    o_ref[...] = lax.dot_general(
        g, w2_ref[...], (((1,), (0,)), ((), ())),
        precision=lax.Precision.HIGHEST,
        preferred_element_type=jnp.float32,
    ) + jnp.float32(N) * b2_ref[...].reshape(1, _H2)


_tc_head = pl.pallas_call(
    _tc_body,
    out_shape=jax.ShapeDtypeStruct((1, _H2), jnp.float32),
)


def kernel(x, edge_index, W1, b1, W2, b2):
    ef = edge_index.reshape(-1)
    degp = _sc_deg(ef)
    cp = _sc_c(ef, degp)
    wp = _sc_w(ef, degp, cp)
    return _tc_head(x, degp, cp, wp, W1, b1, W2, b2)


# speedup vs baseline: 74.1399x; 1.0117x over previous
import functools

import jax
import jax.numpy as jnp
from jax import lax
from jax.experimental import pallas as pl
from jax.experimental.pallas import tpu as pltpu
from jax.experimental.pallas import tpu_sc as plsc

N = 10000
E = 320000
NC = 2
NTILES = 16
NW = NC * NTILES
PN = 10240
OWNT = PN // NTILES
OWNTV = OWNT // 16
EPW = E // NW

_D_IN = 128
_H1 = 256
_H2 = 128


def _rsqrt16(d):
    x = jnp.maximum(d, 1.0)
    i = lax.bitcast_convert_type(x, jnp.int32)
    i = jnp.int32(0x5F3759DF) - lax.shift_right_logical(i, 1)
    y = lax.bitcast_convert_type(i, jnp.float32)
    for _ in range(3):
        y = y * (jnp.float32(1.5) - jnp.float32(0.5) * x * y * y)
    return jnp.where(d > 0.5, y, 0.0)


_MESH = plsc.VectorSubcoreMesh(
    core_axis_name="c", subcore_axis_name="s", num_cores=NC
)


def _fill(ref, nvec, val):
    v = jnp.full((16,), val, jnp.float32)

    def body(j, carry):
        ref[pl.ds(j * 16, 16)] = v
        return carry

    lax.fori_loop(0, nvec, body, 0)


@functools.partial(
    pl.kernel,
    out_type=jax.ShapeDtypeStruct((NC * PN,), jnp.float32),
    mesh=_MESH,
    scratch_types=[
        pltpu.VMEM((EPW,), jnp.int32),
        pltpu.VMEM((EPW,), jnp.float32),
        pltpu.VMEM((OWNT,), jnp.float32),
        pltpu.SemaphoreType.DMA,
        pltpu.VMEM_SHARED((PN,), jnp.float32),
    ],
)
def _sc_deg(edge_hbm, degp_hbm, dstbuf, ones, zbuf, sem, shacc):
    cid = lax.axis_index("c")
    tid = lax.axis_index("s")
    wid = tid * NC + cid
    tile_sl = pl.ds(OWNT * tid, OWNT)

    cp = pltpu.async_copy(edge_hbm.at[pl.ds(E + EPW * wid, EPW)], dstbuf, sem)
    _fill(zbuf, OWNTV, 0.0)
    pltpu.sync_copy(zbuf, shacc.at[tile_sl])
    _fill(ones, EPW // 16, 1.0)
    cp.wait()
    plsc.subcore_barrier()

    pltpu.sync_copy(ones, shacc.at[dstbuf], add=True)
    plsc.subcore_barrier()
    pltpu.sync_copy(shacc.at[tile_sl], degp_hbm.at[pl.ds(PN * cid + OWNT * tid, OWNT)])


HCH = EPW // 2


@functools.partial(
    pl.kernel,
    out_type=jax.ShapeDtypeStruct((NC * PN,), jnp.float32),
    mesh=_MESH,
    scratch_types=[
        pltpu.VMEM((HCH,), jnp.int32),
        pltpu.VMEM((HCH,), jnp.int32),
        pltpu.VMEM((HCH,), jnp.int32),
        pltpu.VMEM((HCH,), jnp.int32),
        pltpu.VMEM((HCH,), jnp.float32),
        pltpu.VMEM((HCH,), jnp.float32),
        pltpu.VMEM((OWNT,), jnp.float32),
        pltpu.VMEM((OWNT,), jnp.float32),
        pltpu.VMEM((OWNT,), jnp.float32),
        pltpu.VMEM((OWNT,), jnp.float32),
        pltpu.SemaphoreType.DMA,
        pltpu.SemaphoreType.DMA,
        pltpu.SemaphoreType.DMA,
        pltpu.VMEM_SHARED((PN,), jnp.float32),
        pltpu.VMEM_SHARED((PN,), jnp.float32),
    ],
)
def _sc_c(edge_hbm, degp_hbm, cp_hbm,
          srcA, srcB, dstA, dstB, gbufA, gbufB, p0, p1, normt, zbuf,
          sem_stage, sem_tbl, sem_g, shacc, shtab):
    cid = lax.axis_index("c")
    tid = lax.axis_index("s")
    wid = tid * NC + cid
    tile_sl = pl.ds(OWNT * tid, OWNT)
    ebase = EPW * wid

    stages = [
        pltpu.async_copy(edge_hbm.at[pl.ds(ebase, HCH)], srcA, sem_stage),
        pltpu.async_copy(edge_hbm.at[pl.ds(ebase + HCH, HCH)], srcB, sem_stage),
        pltpu.async_copy(edge_hbm.at[pl.ds(E + ebase, HCH)], dstA, sem_stage),
        pltpu.async_copy(edge_hbm.at[pl.ds(E + ebase + HCH, HCH)], dstB, sem_stage),
    ]
    tbls = [
        pltpu.async_copy(degp_hbm.at[pl.ds(OWNT * tid, OWNT)], p0, sem_tbl),
        pltpu.async_copy(degp_hbm.at[pl.ds(PN + OWNT * tid, OWNT)], p1, sem_tbl),
    ]
    _fill(zbuf, OWNTV, 0.0)
    pltpu.sync_copy(zbuf, shacc.at[tile_sl])
    for t in tbls:
        t.wait()

    def norm_body(j, carry):
        sl = pl.ds(j * 16, 16)
        normt[sl] = _rsqrt16(p0[sl] + p1[sl])
        return carry

    lax.fori_loop(0, OWNTV, norm_body, 0)
    pltpu.sync_copy(normt, shtab.at[tile_sl])
    for s_ in stages:
        s_.wait()
    plsc.subcore_barrier()

    gA = pltpu.async_copy(shtab.at[dstA], gbufA, sem_g)
    gA.wait()
    gB = pltpu.async_copy(shtab.at[dstB], gbufB, sem_g)
    pltpu.sync_copy(gbufA, shacc.at[srcA], add=True)
    gB.wait()
    pltpu.sync_copy(gbufB, shacc.at[srcB], add=True)
    plsc.subcore_barrier()
    pltpu.sync_copy(shacc.at[tile_sl], cp_hbm.at[pl.ds(PN * cid + OWNT * tid, OWNT)])


@functools.partial(
    pl.kernel,
    out_type=jax.ShapeDtypeStruct((NC * PN,), jnp.float32),
    mesh=_MESH,
    scratch_types=[
        pltpu.VMEM((HCH,), jnp.int32),
        pltpu.VMEM((HCH,), jnp.int32),
        pltpu.VMEM((HCH,), jnp.int32),
        pltpu.VMEM((HCH,), jnp.int32),
        pltpu.VMEM((HCH,), jnp.float32),
        pltpu.VMEM((HCH,), jnp.float32),
        pltpu.VMEM((OWNT,), jnp.float32),
        pltpu.VMEM((OWNT,), jnp.float32),
        pltpu.VMEM((OWNT,), jnp.float32),
        pltpu.VMEM((OWNT,), jnp.float32),
        pltpu.SemaphoreType.DMA,
        pltpu.SemaphoreType.DMA,
        pltpu.SemaphoreType.DMA,
        pltpu.VMEM_SHARED((PN,), jnp.float32),
        pltpu.VMEM_SHARED((PN,), jnp.float32),
    ],
)
def _sc_w(edge_hbm, degp_hbm, cp_hbm, wp_hbm,
          srcA, srcB, dstA, dstB, gbufA, gbufB, p0, p1, tbuf, zbuf,
          sem_stage, sem_tbl, sem_g, shacc, shtab):
    cid = lax.axis_index("c")
    tid = lax.axis_index("s")
    wid = tid * NC + cid
    tile_sl = pl.ds(OWNT * tid, OWNT)
    ebase = EPW * wid

    stages = [
        pltpu.async_copy(edge_hbm.at[pl.ds(ebase, HCH)], srcA, sem_stage),
        pltpu.async_copy(edge_hbm.at[pl.ds(ebase + HCH, HCH)], srcB, sem_stage),
        pltpu.async_copy(edge_hbm.at[pl.ds(E + ebase, HCH)], dstA, sem_stage),
        pltpu.async_copy(edge_hbm.at[pl.ds(E + ebase + HCH, HCH)], dstB, sem_stage),
    ]
    dtbls = [
        pltpu.async_copy(degp_hbm.at[pl.ds(OWNT * tid, OWNT)], p0, sem_tbl),
        pltpu.async_copy(degp_hbm.at[pl.ds(PN + OWNT * tid, OWNT)], p1, sem_tbl),
    ]
    for t in dtbls:
        t.wait()

    def norm_body(j, carry):
        sl = pl.ds(j * 16, 16)
        zbuf[sl] = _rsqrt16(p0[sl] + p1[sl])
        return carry

    lax.fori_loop(0, OWNTV, norm_body, 0)
    ctbls = [
        pltpu.async_copy(cp_hbm.at[pl.ds(OWNT * tid, OWNT)], p0, sem_tbl),
        pltpu.async_copy(cp_hbm.at[pl.ds(PN + OWNT * tid, OWNT)], p1, sem_tbl),
    ]
    for t in ctbls:
        t.wait()

    def t_body(j, carry):
        sl = pl.ds(j * 16, 16)
        nrm = zbuf[sl]
        tbuf[sl] = nrm * nrm * (p0[sl] + p1[sl])
        return carry

    lax.fori_loop(0, OWNTV, t_body, 0)
    pltpu.sync_copy(tbuf, shtab.at[tile_sl])
    _fill(zbuf, OWNTV, 0.0)
    pltpu.sync_copy(zbuf, shacc.at[tile_sl])
    for s_ in stages:
        s_.wait()
    plsc.subcore_barrier()

    gA = pltpu.async_copy(shtab.at[dstA], gbufA, sem_g)
    gA.wait()
    gB = pltpu.async_copy(shtab.at[dstB], gbufB, sem_g)
    pltpu.sync_copy(gbufA, shacc.at[srcA], add=True)
    gB.wait()
    pltpu.sync_copy(gbufB, shacc.at[srcB], add=True)
    plsc.subcore_barrier()
    pltpu.sync_copy(shacc.at[tile_sl], wp_hbm.at[pl.ds(PN * cid + OWNT * tid, OWNT)])


def _tc_body(x_ref, degp_ref, cp_ref, wp_ref, w1_ref, b1_ref, w2_ref, b2_ref,
             o_ref):
    degp = degp_ref[...].reshape(NC, PN)
    cp = cp_ref[...].reshape(NC, PN)
    wp = wp_ref[...].reshape(NC, PN)
    deg = degp[0] + degp[1]
    norm = jnp.where(deg > 0, lax.rsqrt(jnp.maximum(deg, 1.0)), 0.0)
    u = norm * (cp[0] + cp[1])
    w = norm * (wp[0] + wp[1])
    wrow = w.reshape(1, PN)[:, :N]
    r = lax.dot_general(
        wrow, x_ref[...], (((1,), (0,)), ((), ())),
        precision=lax.Precision.HIGHEST,
        preferred_element_type=jnp.float32,
    )
    s = jnp.sum(u)
    g = lax.dot_general(
        r, w1_ref[...], (((1,), (0,)), ((), ())),
        precision=lax.Precision.HIGHEST,
        preferred_element_type=jnp.float32,
    ) + s * b1_ref[...].reshape(1, _H1)
    o_ref[...] = lax.dot_general(
        g, w2_ref[...], (((1,), (0,)), ((), ())),
        precision=lax.Precision.HIGHEST,
        preferred_element_type=jnp.float32,
    ) + jnp.float32(N) * b2_ref[...].reshape(1, _H2)


_tc_head = pl.pallas_call(
    _tc_body,
    out_shape=jax.ShapeDtypeStruct((1, _H2), jnp.float32),
)


def kernel(x, edge_index, W1, b1, W2, b2):
    ef = edge_index.reshape(-1)
    degp = _sc_deg(ef)
    cp = _sc_c(ef, degp)
    wp = _sc_w(ef, degp, cp)
    return _tc_head(x, degp, cp, wp, W1, b1, W2, b2)
